# baseline (speedup 1.0000x reference)
import jax
import jax.numpy as jnp
from jax.experimental import pallas as pl
from jax.experimental.pallas import tpu as pltpu

IN_DIM = 1024
OUT_DIM = 1024
K = 2048
B_TOK = 8192
TM = 512


def _prep_body(a_ref, b_ref, a_bf_ref, b_bf_ref, inv_ref):
    a32 = a_ref[...]
    inv = jax.lax.rsqrt(jnp.sum(a32 * a32, axis=0, keepdims=True))
    inv_ref[...] = inv
    a_bf_ref[...] = a32.astype(jnp.bfloat16)
    b_bf_ref[...] = (b_ref[...] * inv.T).astype(jnp.bfloat16)


def _main_body(x_ref, a_bf_ref, b_bf_ref, inv_ref, out_ref, inner_ref):
    x_bf = x_ref[...].astype(jnp.bfloat16)
    inner_raw = jnp.dot(x_bf, a_bf_ref[...],
                        preferred_element_type=jnp.float32)
    inner_ref[...] = inner_raw * inv_ref[...]
    out_ref[...] = jnp.dot(inner_raw.astype(jnp.bfloat16), b_bf_ref[...],
                           preferred_element_type=jnp.float32)


def kernel(x, A, Bm):
    a_bf, b_bf, inv = pl.pallas_call(
        _prep_body,
        out_shape=[
            jax.ShapeDtypeStruct((IN_DIM, K), jnp.bfloat16),
            jax.ShapeDtypeStruct((K, OUT_DIM), jnp.bfloat16),
            jax.ShapeDtypeStruct((1, K), jnp.float32),
        ],
    )(A, Bm)

    n_tiles = B_TOK // TM
    out, inner = pl.pallas_call(
        _main_body,
        grid=(n_tiles,),
        in_specs=[
            pl.BlockSpec((TM, IN_DIM), lambda i: (i, 0)),
            pl.BlockSpec((IN_DIM, K), lambda i: (0, 0)),
            pl.BlockSpec((K, OUT_DIM), lambda i: (0, 0)),
            pl.BlockSpec((1, K), lambda i: (0, 0)),
        ],
        out_specs=[
            pl.BlockSpec((TM, OUT_DIM), lambda i: (i, 0)),
            pl.BlockSpec((TM, K), lambda i: (i, 0)),
        ],
        out_shape=[
            jax.ShapeDtypeStruct((B_TOK, OUT_DIM), jnp.float32),
            jax.ShapeDtypeStruct((B_TOK, K), jnp.float32),
        ],
        compiler_params=pltpu.CompilerParams(
            dimension_semantics=("parallel",),
        ),
    )(x, a_bf, b_bf, inv)
    return (out, inner)


# speedup vs baseline: 1.0304x; 1.0304x over previous
import jax
import jax.numpy as jnp
from jax.experimental import pallas as pl
from jax.experimental.pallas import tpu as pltpu

IN_DIM = 1024
OUT_DIM = 1024
K = 2048
B_TOK = 8192
TM = 512


def _inv_norm_body(A_ref, inv_ref):
    s = jnp.sum(A_ref[...] * A_ref[...], axis=0, keepdims=True)
    inv_ref[...] = jax.lax.rsqrt(s)


def _fused_body(x_ref, A_ref, B_ref, inv_ref, out_ref, inner_ref):
    inner = jnp.dot(x_ref[...], A_ref[...], preferred_element_type=jnp.float32)
    inner = inner * inv_ref[...]
    inner_ref[...] = inner
    out_ref[...] = jnp.dot(inner, B_ref[...], preferred_element_type=jnp.float32)


def kernel(x, A, Bm):
    inv = pl.pallas_call(
        _inv_norm_body,
        in_specs=[pl.BlockSpec((IN_DIM, K), lambda: (0, 0))],
        out_specs=pl.BlockSpec((1, K), lambda: (0, 0)),
        out_shape=jax.ShapeDtypeStruct((1, K), jnp.float32),
    )(A)

    n_tiles = B_TOK // TM
    out, inner = pl.pallas_call(
        _fused_body,
        grid=(n_tiles,),
        in_specs=[
            pl.BlockSpec((TM, IN_DIM), lambda i: (i, 0)),
            pl.BlockSpec((IN_DIM, K), lambda i: (0, 0)),
            pl.BlockSpec((K, OUT_DIM), lambda i: (0, 0)),
            pl.BlockSpec((1, K), lambda i: (0, 0)),
        ],
        out_specs=[
            pl.BlockSpec((TM, OUT_DIM), lambda i: (i, 0)),
            pl.BlockSpec((TM, K), lambda i: (i, 0)),
        ],
        out_shape=[
            jax.ShapeDtypeStruct((B_TOK, OUT_DIM), jnp.float32),
            jax.ShapeDtypeStruct((B_TOK, K), jnp.float32),
        ],
        compiler_params=pltpu.CompilerParams(
            dimension_semantics=("parallel",),
        ),
    )(x, A, Bm, inv)
    return (out, inner)
